# baseline (speedup 1.0000x reference)
import jax
import jax.numpy as jnp
from jax.experimental import pallas as pl

_W = 0.3
_BLOCK_ROWS = 2000


def _fuse_kernel(cu_ref, ci_ref, hu_ref, hi_ref, ou_ref, oi_ref):
    ou_ref[...] = (1.0 - _W) * cu_ref[...] + _W * hu_ref[...]
    oi_ref[...] = (1.0 - _W) * ci_ref[...] + _W * hi_ref[...]


def kernel(core_user_emb, core_item_emb, hg_user_emb, hg_item_emb):
    M, D = core_user_emb.shape
    grid = (M // _BLOCK_ROWS,)
    spec = pl.BlockSpec((_BLOCK_ROWS, D), lambda i: (i, 0))
    out_user, out_item = pl.pallas_call(
        _fuse_kernel,
        grid=grid,
        in_specs=[spec, spec, spec, spec],
        out_specs=[spec, spec],
        out_shape=[
            jax.ShapeDtypeStruct((M, D), core_user_emb.dtype),
            jax.ShapeDtypeStruct((M, D), core_item_emb.dtype),
        ],
    )(core_user_emb, core_item_emb, hg_user_emb, hg_item_emb)
    return (out_user, out_item)


# speedup vs baseline: 1.0312x; 1.0312x over previous
import jax
import jax.numpy as jnp
from jax.experimental import pallas as pl

_W = 0.3
_BLOCK_ROWS = 5000


def _fuse_kernel(cu_ref, ci_ref, hu_ref, hi_ref, ou_ref, oi_ref):
    ou_ref[...] = (1.0 - _W) * cu_ref[...] + _W * hu_ref[...]
    oi_ref[...] = (1.0 - _W) * ci_ref[...] + _W * hi_ref[...]


def kernel(core_user_emb, core_item_emb, hg_user_emb, hg_item_emb):
    M, D = core_user_emb.shape
    grid = (M // _BLOCK_ROWS,)
    spec = pl.BlockSpec((_BLOCK_ROWS, D), lambda i: (i, 0))
    out_user, out_item = pl.pallas_call(
        _fuse_kernel,
        grid=grid,
        in_specs=[spec, spec, spec, spec],
        out_specs=[spec, spec],
        out_shape=[
            jax.ShapeDtypeStruct((M, D), core_user_emb.dtype),
            jax.ShapeDtypeStruct((M, D), core_item_emb.dtype),
        ],
    )(core_user_emb, core_item_emb, hg_user_emb, hg_item_emb)
    return (out_user, out_item)


# speedup vs baseline: 1.0369x; 1.0055x over previous
import jax
import jax.numpy as jnp
from jax.experimental import pallas as pl

_W = 0.3
_BLOCK_ROWS = 4000


def _fuse_kernel(cu_ref, ci_ref, hu_ref, hi_ref, ou_ref, oi_ref):
    ou_ref[...] = (1.0 - _W) * cu_ref[...] + _W * hu_ref[...]
    oi_ref[...] = (1.0 - _W) * ci_ref[...] + _W * hi_ref[...]


def kernel(core_user_emb, core_item_emb, hg_user_emb, hg_item_emb):
    M, D = core_user_emb.shape
    grid = (M // _BLOCK_ROWS,)
    spec = pl.BlockSpec((_BLOCK_ROWS, D), lambda i: (i, 0))
    out_user, out_item = pl.pallas_call(
        _fuse_kernel,
        grid=grid,
        in_specs=[spec, spec, spec, spec],
        out_specs=[spec, spec],
        out_shape=[
            jax.ShapeDtypeStruct((M, D), core_user_emb.dtype),
            jax.ShapeDtypeStruct((M, D), core_item_emb.dtype),
        ],
    )(core_user_emb, core_item_emb, hg_user_emb, hg_item_emb)
    return (out_user, out_item)
